# Initial kernel scaffold; baseline (speedup 1.0000x reference)
import jax
import jax.numpy as jnp
from jax.experimental import pallas as pl


def kernel(x, W1, b1, W2, b2, g1, be1, g2, be2, Wf1, bf1, Wf2, bf2):
    raise NotImplementedError("write your pallas kernel here")


# speedup vs baseline: 2.4773x; 2.4773x over previous
import functools

import jax
import jax.numpy as jnp
from jax.experimental import pallas as pl
from jax.experimental.pallas import tpu as pltpu

_TB = 256
_VMEM = 96 * 1024 * 1024


def _stage1_body(x_ref, w_ref, b_ref, z_ref, st_ref, *, n_valid, tb):
    w = w_ref[...]
    chunks = []
    for blk in range(6):
        y = jnp.dot(x_ref[:, 128 * blk:128 * blk + 256], w,
                    preferred_element_type=jnp.float32)
        p = jnp.maximum(jnp.maximum(y[:, 0:192], y[:, 192:384]),
                        jnp.maximum(y[:, 384:576], y[:, 576:768]))
        chunks.append(p)
    z = jnp.concatenate(chunks, axis=-1)
    z = jnp.maximum(z + b_ref[...], 0.0).astype(z_ref.dtype)
    z_ref[...] = z

    zf = z.astype(jnp.float32)
    if n_valid is not None:
        row = jax.lax.broadcasted_iota(jnp.int32, (tb, 1), 0) + pl.program_id(0) * tb
        zf = jnp.where(row < n_valid, zf, 0.0)
    s = jnp.sum(zf, axis=0, keepdims=True)
    sq = jnp.sum(zf * zf, axis=0, keepdims=True)
    st_ref[...] = jnp.concatenate([s, sq], axis=0)[None]


def _stage2_body(z1_ref, w_ref, b_ref, z_ref, st_ref, *, n_valid, tb):
    zn = z1_ref[...]
    w = w_ref[...]
    chunks = []
    for blk in range(2):
        y = jnp.dot(zn[:, 384 * blk:384 * blk + 768], w,
                    preferred_element_type=jnp.float32)
        p = jnp.maximum(jnp.maximum(y[:, 0:64], y[:, 64:128]),
                        jnp.maximum(y[:, 128:192], y[:, 192:256]))
        chunks.append(p)
    z = jnp.concatenate(chunks, axis=-1)
    z = jnp.maximum(z + b_ref[...], 0.0).astype(z_ref.dtype)
    z_ref[...] = z

    zf = z.astype(jnp.float32)
    if n_valid is not None:
        row = jax.lax.broadcasted_iota(jnp.int32, (tb, 1), 0) + pl.program_id(0) * tb
        zf = jnp.where(row < n_valid, zf, 0.0)
    s = jnp.sum(zf, axis=0, keepdims=True)
    sq = jnp.sum(zf * zf, axis=0, keepdims=True)
    st_ref[...] = jnp.concatenate([s, sq], axis=0)[None]


def _head_body(z2_ref, w1_ref, b1_ref, w2_ref, b2_ref, o_ref):
    h = jnp.dot(z2_ref[...], w1_ref[...], preferred_element_type=jnp.float32)
    h = jnp.maximum(h + b1_ref[...], 0.0)
    logits = jnp.dot(h.astype(jnp.bfloat16), w2_ref[...],
                     preferred_element_type=jnp.float32) + b2_ref[...]
    m = jnp.max(logits, axis=-1, keepdims=True)
    lse = jnp.log(jnp.sum(jnp.exp(logits - m), axis=-1, keepdims=True)) + m
    o_ref[...] = (logits - lse).astype(o_ref.dtype)


def _band_block1(w):
    l = jnp.arange(8); wi = jnp.arange(32)
    phl = jnp.arange(2); rp = jnp.arange(2)
    pw = jnp.arange(12); wp = jnp.arange(2)
    kh = l[:, None, None] - 2 * phl[None, :, None] - rp[None, None, :]
    kw = wi[:, None, None] - 2 * pw[None, :, None] - wp[None, None, :]
    vh = (kh >= 0) & (kh < 5)
    vw = (kw >= 0) & (kw < 5)
    wc = w[:, 0]
    t = wc[:, kh.clip(0, 4), :]
    t = t[..., kw.clip(0, 4)]
    mask = (vh[None, :, :, :, None, None, None]
            & vw[None, None, None, None, :, :, :])
    t = t * mask.astype(w.dtype)
    t = jnp.transpose(t, (1, 4, 3, 6, 2, 0, 5))
    return t.reshape(256, 768)


def _band_block2(w):
    l = jnp.arange(8); wi = jnp.arange(12)
    phl = jnp.arange(2); rp = jnp.arange(2)
    pw = jnp.arange(4); wp = jnp.arange(2)
    kh = l[:, None, None] - 2 * phl[None, :, None] - rp[None, None, :]
    kw = wi[:, None, None] - 2 * pw[None, :, None] - wp[None, None, :]
    vh = (kh >= 0) & (kh < 5)
    vw = (kw >= 0) & (kw < 5)
    t = w[:, :, kh.clip(0, 4), :]
    t = t[..., kw.clip(0, 4)]
    mask = (vh[None, None, :, :, :, None, None, None]
            & vw[None, None, None, None, None, :, :, :])
    t = t * mask.astype(w.dtype)
    t = jnp.transpose(t, (2, 1, 5, 4, 7, 3, 0, 6))
    return t.reshape(768, 256)


def _per_lane(v, w_rep, h_rep):
    return jnp.tile(jnp.repeat(v, w_rep), h_rep)


def _bn_scale_shift(st, n_valid, h_dim, w_dim, gamma, beta, eps=1e-5):
    tot = st.sum(axis=0)
    per_c = tot.reshape(2, h_dim, 8, w_dim).sum(axis=(1, 3))
    count = n_valid * h_dim * w_dim
    mean = per_c[0] / count
    var = per_c[1] / count - mean * mean
    scale = gamma * jax.lax.rsqrt(var + eps)
    shift = beta - mean * scale
    return scale, shift


def _cdiv(a, b):
    return -(-a // b)


def kernel(x, W1, b1, W2, b2, g1, be1, g2, be2, Wf1, bf1, Wf2, bf2):
    n = x.shape[0]
    tb = min(_TB, _cdiv(n, 8) * 8)
    n_pad = tb * _cdiv(n, tb)
    grid = n_pad // tb
    n_valid = None if n_pad == n else n
    cp = pltpu.CompilerParams(dimension_semantics=("parallel",),
                              vmem_limit_bytes=_VMEM)

    xr = x.reshape(n, 28, 28)
    xr = jnp.pad(xr, ((0, n_pad - n), (0, 0), (0, 4)))
    xb = xr.reshape(n_pad, 896).astype(jnp.bfloat16)

    w1b = _band_block1(W1).astype(jnp.bfloat16)
    b1v = _per_lane(b1, 12, 12)[None].astype(jnp.float32)
    z1, st1 = pl.pallas_call(
        functools.partial(_stage1_body, n_valid=n_valid, tb=tb),
        grid=(grid,),
        in_specs=[
            pl.BlockSpec((tb, 896), lambda i: (i, 0)),
            pl.BlockSpec((256, 768), lambda i: (0, 0)),
            pl.BlockSpec((1, 1152), lambda i: (0, 0)),
        ],
        out_specs=(
            pl.BlockSpec((tb, 1152), lambda i: (i, 0)),
            pl.BlockSpec((1, 2, 1152), lambda i: (i, 0, 0)),
        ),
        out_shape=(
            jax.ShapeDtypeStruct((n_pad, 1152), jnp.bfloat16),
            jax.ShapeDtypeStruct((grid, 2, 1152), jnp.float32),
        ),
        compiler_params=cp,
    )(xb, w1b, b1v)

    s1, t1 = _bn_scale_shift(st1, n, 12, 12, g1, be1)

    w2b = _band_block2(W2)
    w2b = (w2b * jnp.tile(jnp.repeat(s1, 12), 8)[:, None]).astype(jnp.bfloat16)
    b2_eff = b2 + W2.sum(axis=(2, 3)) @ t1
    b2v = _per_lane(b2_eff, 4, 4)[None].astype(jnp.float32)
    z2, st2 = pl.pallas_call(
        functools.partial(_stage2_body, n_valid=n_valid, tb=tb),
        grid=(grid,),
        in_specs=[
            pl.BlockSpec((tb, 1152), lambda i: (i, 0)),
            pl.BlockSpec((768, 256), lambda i: (0, 0)),
            pl.BlockSpec((1, 128), lambda i: (0, 0)),
        ],
        out_specs=(
            pl.BlockSpec((tb, 128), lambda i: (i, 0)),
            pl.BlockSpec((1, 2, 128), lambda i: (i, 0, 0)),
        ),
        out_shape=(
            jax.ShapeDtypeStruct((n_pad, 128), jnp.bfloat16),
            jax.ShapeDtypeStruct((grid, 2, 128), jnp.float32),
        ),
        compiler_params=cp,
    )(z1, w2b, b2v)

    s2, t2 = _bn_scale_shift(st2, n, 4, 4, g2, be2)

    w1h = Wf1.reshape(64, 8, 4, 4).transpose(0, 2, 1, 3).reshape(64, 128).T
    w1h = w1h * _per_lane(s2, 4, 4)[:, None]
    w1p = jnp.pad(w1h, ((0, 0), (0, 64))).astype(jnp.bfloat16)
    b1h = bf1 + Wf1.reshape(64, 8, 16).sum(axis=-1) @ t2
    b1p = jnp.pad(b1h, (0, 64))[None].astype(jnp.float32)
    w2p = jnp.pad(Wf2.T, ((0, 64), (0, 118))).astype(jnp.bfloat16)
    b2p = jnp.concatenate(
        [bf2.astype(jnp.float32), jnp.full((118,), -1e30, jnp.float32)])[None]
    out = pl.pallas_call(
        _head_body,
        grid=(grid,),
        in_specs=[
            pl.BlockSpec((tb, 128), lambda i: (i, 0)),
            pl.BlockSpec((128, 128), lambda i: (0, 0)),
            pl.BlockSpec((1, 128), lambda i: (0, 0)),
            pl.BlockSpec((128, 128), lambda i: (0, 0)),
            pl.BlockSpec((1, 128), lambda i: (0, 0)),
        ],
        out_specs=pl.BlockSpec((tb, 128), lambda i: (i, 0)),
        out_shape=jax.ShapeDtypeStruct((n_pad, 128), jnp.float32),
        compiler_params=cp,
    )(z2, w1p, b1p, w2p, b2p)
    return out[:n, :10]


# speedup vs baseline: 2.7608x; 1.1145x over previous
import functools

import jax
import jax.numpy as jnp
from jax.experimental import pallas as pl
from jax.experimental.pallas import tpu as pltpu

_TB = 512
_VMEM = 96 * 1024 * 1024


def _stage1_body(x_ref, w_ref, b_ref, z_ref, st_ref, *, n_valid, tb):
    w = w_ref[...]
    chunks = []
    for blk in range(6):
        y = jnp.dot(x_ref[:, 128 * blk:128 * blk + 256], w,
                    preferred_element_type=jnp.float32)
        p = jnp.maximum(jnp.maximum(y[:, 0:192], y[:, 192:384]),
                        jnp.maximum(y[:, 384:576], y[:, 576:768]))
        chunks.append(p)
    z = jnp.concatenate(chunks, axis=-1)
    z = jnp.maximum(z + b_ref[...], 0.0).astype(z_ref.dtype)
    z_ref[...] = z

    zf = z.astype(jnp.float32)
    if n_valid is not None:
        row = jax.lax.broadcasted_iota(jnp.int32, (tb, 1), 0) + pl.program_id(0) * tb
        zf = jnp.where(row < n_valid, zf, 0.0)
    s = jnp.sum(zf, axis=0, keepdims=True)
    sq = jnp.sum(zf * zf, axis=0, keepdims=True)
    st_ref[...] = jnp.concatenate([s, sq], axis=0)[None]


def _stage2_body(z1_ref, w_ref, b_ref, z_ref, st_ref, *, n_valid, tb):
    zn = z1_ref[...]
    w = w_ref[...]
    chunks = []
    for blk in range(2):
        y = jnp.dot(zn[:, 384 * blk:384 * blk + 768], w,
                    preferred_element_type=jnp.float32)
        p = jnp.maximum(jnp.maximum(y[:, 0:64], y[:, 64:128]),
                        jnp.maximum(y[:, 128:192], y[:, 192:256]))
        chunks.append(p)
    z = jnp.concatenate(chunks, axis=-1)
    z = jnp.maximum(z + b_ref[...], 0.0).astype(z_ref.dtype)
    z_ref[...] = z

    zf = z.astype(jnp.float32)
    if n_valid is not None:
        row = jax.lax.broadcasted_iota(jnp.int32, (tb, 1), 0) + pl.program_id(0) * tb
        zf = jnp.where(row < n_valid, zf, 0.0)
    s = jnp.sum(zf, axis=0, keepdims=True)
    sq = jnp.sum(zf * zf, axis=0, keepdims=True)
    st_ref[...] = jnp.concatenate([s, sq], axis=0)[None]


def _head_body(z2_ref, w1_ref, b1_ref, w2_ref, b2_ref, o_ref):
    h = jnp.dot(z2_ref[...], w1_ref[...], preferred_element_type=jnp.float32)
    h = jnp.maximum(h + b1_ref[...], 0.0)
    logits = jnp.dot(h.astype(jnp.bfloat16), w2_ref[...],
                     preferred_element_type=jnp.float32) + b2_ref[...]
    m = jnp.max(logits, axis=-1, keepdims=True)
    lse = jnp.log(jnp.sum(jnp.exp(logits - m), axis=-1, keepdims=True)) + m
    o_ref[...] = (logits - lse).astype(o_ref.dtype)


def _band_block1(w):
    l = jnp.arange(8); wi = jnp.arange(32)
    phl = jnp.arange(2); rp = jnp.arange(2)
    pw = jnp.arange(12); wp = jnp.arange(2)
    kh = l[:, None, None] - 2 * phl[None, :, None] - rp[None, None, :]
    kw = wi[:, None, None] - 2 * pw[None, :, None] - wp[None, None, :]
    vh = (kh >= 0) & (kh < 5)
    vw = (kw >= 0) & (kw < 5)
    wc = w[:, 0]
    t = wc[:, kh.clip(0, 4), :]
    t = t[..., kw.clip(0, 4)]
    mask = (vh[None, :, :, :, None, None, None]
            & vw[None, None, None, None, :, :, :])
    t = t * mask.astype(w.dtype)
    t = jnp.transpose(t, (1, 4, 3, 6, 2, 0, 5))
    return t.reshape(256, 768)


def _band_block2(w):
    l = jnp.arange(8); wi = jnp.arange(12)
    phl = jnp.arange(2); rp = jnp.arange(2)
    pw = jnp.arange(4); wp = jnp.arange(2)
    kh = l[:, None, None] - 2 * phl[None, :, None] - rp[None, None, :]
    kw = wi[:, None, None] - 2 * pw[None, :, None] - wp[None, None, :]
    vh = (kh >= 0) & (kh < 5)
    vw = (kw >= 0) & (kw < 5)
    t = w[:, :, kh.clip(0, 4), :]
    t = t[..., kw.clip(0, 4)]
    mask = (vh[None, None, :, :, :, None, None, None]
            & vw[None, None, None, None, None, :, :, :])
    t = t * mask.astype(w.dtype)
    t = jnp.transpose(t, (2, 1, 5, 4, 7, 3, 0, 6))
    return t.reshape(768, 256)


def _per_lane(v, w_rep, h_rep):
    return jnp.tile(jnp.repeat(v, w_rep), h_rep)


def _bn_scale_shift(st, n_valid, h_dim, w_dim, gamma, beta, eps=1e-5):
    tot = st.sum(axis=0)
    per_c = tot.reshape(2, h_dim, 8, w_dim).sum(axis=(1, 3))
    count = n_valid * h_dim * w_dim
    mean = per_c[0] / count
    var = per_c[1] / count - mean * mean
    scale = gamma * jax.lax.rsqrt(var + eps)
    shift = beta - mean * scale
    return scale, shift


def _cdiv(a, b):
    return -(-a // b)


def kernel(x, W1, b1, W2, b2, g1, be1, g2, be2, Wf1, bf1, Wf2, bf2):
    n = x.shape[0]
    tb = min(_TB, _cdiv(n, 8) * 8)
    n_pad = tb * _cdiv(n, tb)
    grid = n_pad // tb
    n_valid = None if n_pad == n else n
    cp = pltpu.CompilerParams(dimension_semantics=("parallel",),
                              vmem_limit_bytes=_VMEM)

    xr = x.reshape(n, 28, 28)
    xr = jnp.pad(xr, ((0, n_pad - n), (0, 0), (0, 4)))
    xb = xr.reshape(n_pad, 896).astype(jnp.bfloat16)

    w1b = _band_block1(W1).astype(jnp.bfloat16)
    b1v = _per_lane(b1, 12, 12)[None].astype(jnp.float32)
    z1, st1 = pl.pallas_call(
        functools.partial(_stage1_body, n_valid=n_valid, tb=tb),
        grid=(grid,),
        in_specs=[
            pl.BlockSpec((tb, 896), lambda i: (i, 0)),
            pl.BlockSpec((256, 768), lambda i: (0, 0)),
            pl.BlockSpec((1, 1152), lambda i: (0, 0)),
        ],
        out_specs=(
            pl.BlockSpec((tb, 1152), lambda i: (i, 0)),
            pl.BlockSpec((1, 2, 1152), lambda i: (i, 0, 0)),
        ),
        out_shape=(
            jax.ShapeDtypeStruct((n_pad, 1152), jnp.bfloat16),
            jax.ShapeDtypeStruct((grid, 2, 1152), jnp.float32),
        ),
        compiler_params=cp,
    )(xb, w1b, b1v)

    s1, t1 = _bn_scale_shift(st1, n, 12, 12, g1, be1)

    w2b = _band_block2(W2)
    w2b = (w2b * jnp.tile(jnp.repeat(s1, 12), 8)[:, None]).astype(jnp.bfloat16)
    b2_eff = b2 + W2.sum(axis=(2, 3)) @ t1
    b2v = _per_lane(b2_eff, 4, 4)[None].astype(jnp.float32)
    z2, st2 = pl.pallas_call(
        functools.partial(_stage2_body, n_valid=n_valid, tb=tb),
        grid=(grid,),
        in_specs=[
            pl.BlockSpec((tb, 1152), lambda i: (i, 0)),
            pl.BlockSpec((768, 256), lambda i: (0, 0)),
            pl.BlockSpec((1, 128), lambda i: (0, 0)),
        ],
        out_specs=(
            pl.BlockSpec((tb, 128), lambda i: (i, 0)),
            pl.BlockSpec((1, 2, 128), lambda i: (i, 0, 0)),
        ),
        out_shape=(
            jax.ShapeDtypeStruct((n_pad, 128), jnp.bfloat16),
            jax.ShapeDtypeStruct((grid, 2, 128), jnp.float32),
        ),
        compiler_params=cp,
    )(z1, w2b, b2v)

    s2, t2 = _bn_scale_shift(st2, n, 4, 4, g2, be2)

    w1h = Wf1.reshape(64, 8, 4, 4).transpose(0, 2, 1, 3).reshape(64, 128).T
    w1h = w1h * _per_lane(s2, 4, 4)[:, None]
    w1p = jnp.pad(w1h, ((0, 0), (0, 64))).astype(jnp.bfloat16)
    b1h = bf1 + Wf1.reshape(64, 8, 16).sum(axis=-1) @ t2
    b1p = jnp.pad(b1h, (0, 64))[None].astype(jnp.float32)
    w2p = jnp.pad(Wf2.T, ((0, 64), (0, 118))).astype(jnp.bfloat16)
    b2p = jnp.concatenate(
        [bf2.astype(jnp.float32), jnp.full((118,), -1e30, jnp.float32)])[None]
    out = pl.pallas_call(
        _head_body,
        grid=(grid,),
        in_specs=[
            pl.BlockSpec((tb, 128), lambda i: (i, 0)),
            pl.BlockSpec((128, 128), lambda i: (0, 0)),
            pl.BlockSpec((1, 128), lambda i: (0, 0)),
            pl.BlockSpec((128, 128), lambda i: (0, 0)),
            pl.BlockSpec((1, 128), lambda i: (0, 0)),
        ],
        out_specs=pl.BlockSpec((tb, 128), lambda i: (i, 0)),
        out_shape=jax.ShapeDtypeStruct((n_pad, 128), jnp.float32),
        compiler_params=cp,
    )(z2, w1p, b1p, w2p, b2p)
    return out[:n, :10]


# speedup vs baseline: 2.9180x; 1.0569x over previous
import functools

import jax
import jax.numpy as jnp
from jax.experimental import pallas as pl
from jax.experimental.pallas import tpu as pltpu

_TB = 1024
_VMEM = 96 * 1024 * 1024


def _stage1_body(x_ref, w_ref, b_ref, z_ref, st_ref, *, n_valid, tb):
    w = w_ref[...]
    chunks = []
    for blk in range(6):
        y = jnp.dot(x_ref[:, 128 * blk:128 * blk + 256], w,
                    preferred_element_type=jnp.float32)
        p = jnp.maximum(jnp.maximum(y[:, 0:192], y[:, 192:384]),
                        jnp.maximum(y[:, 384:576], y[:, 576:768]))
        chunks.append(p)
    z = jnp.concatenate(chunks, axis=-1)
    z = jnp.maximum(z + b_ref[...], 0.0).astype(z_ref.dtype)
    z_ref[...] = z

    zf = z.astype(jnp.float32)
    if n_valid is not None:
        row = jax.lax.broadcasted_iota(jnp.int32, (tb, 1), 0) + pl.program_id(0) * tb
        zf = jnp.where(row < n_valid, zf, 0.0)
    s = jnp.sum(zf, axis=0, keepdims=True)
    sq = jnp.sum(zf * zf, axis=0, keepdims=True)
    st_ref[...] = jnp.concatenate([s, sq], axis=0)[None]


def _stage2_body(z1_ref, w_ref, b_ref, z_ref, st_ref, *, n_valid, tb):
    zn = z1_ref[...]
    w = w_ref[...]
    chunks = []
    for blk in range(2):
        y = jnp.dot(zn[:, 384 * blk:384 * blk + 768], w,
                    preferred_element_type=jnp.float32)
        p = jnp.maximum(jnp.maximum(y[:, 0:64], y[:, 64:128]),
                        jnp.maximum(y[:, 128:192], y[:, 192:256]))
        chunks.append(p)
    z = jnp.concatenate(chunks, axis=-1)
    z = jnp.maximum(z + b_ref[...], 0.0).astype(z_ref.dtype)
    z_ref[...] = z

    zf = z.astype(jnp.float32)
    if n_valid is not None:
        row = jax.lax.broadcasted_iota(jnp.int32, (tb, 1), 0) + pl.program_id(0) * tb
        zf = jnp.where(row < n_valid, zf, 0.0)
    s = jnp.sum(zf, axis=0, keepdims=True)
    sq = jnp.sum(zf * zf, axis=0, keepdims=True)
    st_ref[...] = jnp.concatenate([s, sq], axis=0)[None]


def _head_body(z2_ref, w1_ref, b1_ref, w2_ref, b2_ref, o_ref):
    h = jnp.dot(z2_ref[...], w1_ref[...], preferred_element_type=jnp.float32)
    h = jnp.maximum(h + b1_ref[...], 0.0)
    logits = jnp.dot(h.astype(jnp.bfloat16), w2_ref[...],
                     preferred_element_type=jnp.float32) + b2_ref[...]
    m = jnp.max(logits, axis=-1, keepdims=True)
    lse = jnp.log(jnp.sum(jnp.exp(logits - m), axis=-1, keepdims=True)) + m
    o_ref[...] = (logits - lse).astype(o_ref.dtype)


def _band_block1(w):
    l = jnp.arange(8); wi = jnp.arange(32)
    phl = jnp.arange(2); rp = jnp.arange(2)
    pw = jnp.arange(12); wp = jnp.arange(2)
    kh = l[:, None, None] - 2 * phl[None, :, None] - rp[None, None, :]
    kw = wi[:, None, None] - 2 * pw[None, :, None] - wp[None, None, :]
    vh = (kh >= 0) & (kh < 5)
    vw = (kw >= 0) & (kw < 5)
    wc = w[:, 0]
    t = wc[:, kh.clip(0, 4), :]
    t = t[..., kw.clip(0, 4)]
    mask = (vh[None, :, :, :, None, None, None]
            & vw[None, None, None, None, :, :, :])
    t = t * mask.astype(w.dtype)
    t = jnp.transpose(t, (1, 4, 3, 6, 2, 0, 5))
    return t.reshape(256, 768)


def _band_block2(w):
    l = jnp.arange(8); wi = jnp.arange(12)
    phl = jnp.arange(2); rp = jnp.arange(2)
    pw = jnp.arange(4); wp = jnp.arange(2)
    kh = l[:, None, None] - 2 * phl[None, :, None] - rp[None, None, :]
    kw = wi[:, None, None] - 2 * pw[None, :, None] - wp[None, None, :]
    vh = (kh >= 0) & (kh < 5)
    vw = (kw >= 0) & (kw < 5)
    t = w[:, :, kh.clip(0, 4), :]
    t = t[..., kw.clip(0, 4)]
    mask = (vh[None, None, :, :, :, None, None, None]
            & vw[None, None, None, None, None, :, :, :])
    t = t * mask.astype(w.dtype)
    t = jnp.transpose(t, (2, 1, 5, 4, 7, 3, 0, 6))
    return t.reshape(768, 256)


def _per_lane(v, w_rep, h_rep):
    return jnp.tile(jnp.repeat(v, w_rep), h_rep)


def _bn_scale_shift(st, n_valid, h_dim, w_dim, gamma, beta, eps=1e-5):
    tot = st.sum(axis=0)
    per_c = tot.reshape(2, h_dim, 8, w_dim).sum(axis=(1, 3))
    count = n_valid * h_dim * w_dim
    mean = per_c[0] / count
    var = per_c[1] / count - mean * mean
    scale = gamma * jax.lax.rsqrt(var + eps)
    shift = beta - mean * scale
    return scale, shift


def _cdiv(a, b):
    return -(-a // b)


def kernel(x, W1, b1, W2, b2, g1, be1, g2, be2, Wf1, bf1, Wf2, bf2):
    n = x.shape[0]
    tb = min(_TB, _cdiv(n, 8) * 8)
    n_pad = tb * _cdiv(n, tb)
    grid = n_pad // tb
    n_valid = None if n_pad == n else n
    cp = pltpu.CompilerParams(dimension_semantics=("parallel",),
                              vmem_limit_bytes=_VMEM)

    xr = x.reshape(n, 28, 28)
    xr = jnp.pad(xr, ((0, n_pad - n), (0, 0), (0, 4)))
    xb = xr.reshape(n_pad, 896).astype(jnp.bfloat16)

    w1b = _band_block1(W1).astype(jnp.bfloat16)
    b1v = _per_lane(b1, 12, 12)[None].astype(jnp.float32)
    z1, st1 = pl.pallas_call(
        functools.partial(_stage1_body, n_valid=n_valid, tb=tb),
        grid=(grid,),
        in_specs=[
            pl.BlockSpec((tb, 896), lambda i: (i, 0)),
            pl.BlockSpec((256, 768), lambda i: (0, 0)),
            pl.BlockSpec((1, 1152), lambda i: (0, 0)),
        ],
        out_specs=(
            pl.BlockSpec((tb, 1152), lambda i: (i, 0)),
            pl.BlockSpec((1, 2, 1152), lambda i: (i, 0, 0)),
        ),
        out_shape=(
            jax.ShapeDtypeStruct((n_pad, 1152), jnp.bfloat16),
            jax.ShapeDtypeStruct((grid, 2, 1152), jnp.float32),
        ),
        compiler_params=cp,
    )(xb, w1b, b1v)

    s1, t1 = _bn_scale_shift(st1, n, 12, 12, g1, be1)

    w2b = _band_block2(W2)
    w2b = (w2b * jnp.tile(jnp.repeat(s1, 12), 8)[:, None]).astype(jnp.bfloat16)
    b2_eff = b2 + W2.sum(axis=(2, 3)) @ t1
    b2v = _per_lane(b2_eff, 4, 4)[None].astype(jnp.float32)
    z2, st2 = pl.pallas_call(
        functools.partial(_stage2_body, n_valid=n_valid, tb=tb),
        grid=(grid,),
        in_specs=[
            pl.BlockSpec((tb, 1152), lambda i: (i, 0)),
            pl.BlockSpec((768, 256), lambda i: (0, 0)),
            pl.BlockSpec((1, 128), lambda i: (0, 0)),
        ],
        out_specs=(
            pl.BlockSpec((tb, 128), lambda i: (i, 0)),
            pl.BlockSpec((1, 2, 128), lambda i: (i, 0, 0)),
        ),
        out_shape=(
            jax.ShapeDtypeStruct((n_pad, 128), jnp.bfloat16),
            jax.ShapeDtypeStruct((grid, 2, 128), jnp.float32),
        ),
        compiler_params=cp,
    )(z1, w2b, b2v)

    s2, t2 = _bn_scale_shift(st2, n, 4, 4, g2, be2)

    w1h = Wf1.reshape(64, 8, 4, 4).transpose(0, 2, 1, 3).reshape(64, 128).T
    w1h = w1h * _per_lane(s2, 4, 4)[:, None]
    w1p = jnp.pad(w1h, ((0, 0), (0, 64))).astype(jnp.bfloat16)
    b1h = bf1 + Wf1.reshape(64, 8, 16).sum(axis=-1) @ t2
    b1p = jnp.pad(b1h, (0, 64))[None].astype(jnp.float32)
    w2p = jnp.pad(Wf2.T, ((0, 64), (0, 118))).astype(jnp.bfloat16)
    b2p = jnp.concatenate(
        [bf2.astype(jnp.float32), jnp.full((118,), -1e30, jnp.float32)])[None]
    out = pl.pallas_call(
        _head_body,
        grid=(grid,),
        in_specs=[
            pl.BlockSpec((tb, 128), lambda i: (i, 0)),
            pl.BlockSpec((128, 128), lambda i: (0, 0)),
            pl.BlockSpec((1, 128), lambda i: (0, 0)),
            pl.BlockSpec((128, 128), lambda i: (0, 0)),
            pl.BlockSpec((1, 128), lambda i: (0, 0)),
        ],
        out_specs=pl.BlockSpec((tb, 128), lambda i: (i, 0)),
        out_shape=jax.ShapeDtypeStruct((n_pad, 128), jnp.float32),
        compiler_params=cp,
    )(z2, w1p, b1p, w2p, b2p)
    return out[:n, :10]
